# Initial kernel scaffold; baseline (speedup 1.0000x reference)
import jax
import jax.numpy as jnp
from jax.experimental import pallas as pl


def kernel(mem, val, step, batch_idx):
    raise NotImplementedError("write your pallas kernel here")


# speedup vs baseline: 3.0489x; 3.0489x over previous
import functools

import jax
import jax.numpy as jnp
from jax import lax
from jax.experimental import pallas as pl
from jax.experimental.pallas import tpu as pltpu
from jax.experimental.pallas import tpu_sc as plsc

T = 32
B = 4096
D = 128
TB = T * B
MB_ROWS = TB // 4

NC = 2
NS = 16
NW = NC * NS
N_PER_W = MB_ROWS // NW
C = 256
NCHUNK = N_PER_W // C
NBUF = 3
LANES = 16
IGNORED = -1


def _body(mem_hbm, val_hbm, lo_hbm, idx_hbm, out_hbm,
          idxm_v, idxv_v, lo_v, buf0, buf1, buf2,
          gm0, gm1, gm2, gv0, gv1, gv2, ss0, ss1, ss2):
    bufs = (buf0, buf1, buf2)
    gm_sems = (gm0, gm1, gm2)
    gv_sems = (gv0, gv1, gv2)
    st_sems = (ss0, ss1, ss2)

    wid = lax.axis_index("s") * NC + lax.axis_index("c")
    base = wid * N_PER_W

    pltpu.sync_copy(idx_hbm.at[pl.ds(base, N_PER_W)], idxm_v)
    pltpu.sync_copy(lo_hbm, lo_v)
    lo = lo_v[...]

    def split(j, carry):
        iv = idxm_v[pl.ds(j * LANES, LANES)]
        m = (iv >= lo) & (iv < lo + B)
        idxm_v[pl.ds(j * LANES, LANES)] = jnp.where(m, IGNORED, iv)
        idxv_v[pl.ds(j * LANES, LANES)] = jnp.where(m, iv - lo, IGNORED)
        return carry

    lax.fori_loop(0, N_PER_W // LANES, split, 0)

    def start_gathers(c):
        bi = c % NBUF
        sl = pl.ds(c * C, C)
        gm = pltpu.async_copy(
            mem_hbm.at[plsc.Indices(idxm_v.at[sl], ignored_value=IGNORED)],
            bufs[bi], gm_sems[bi])
        gv = pltpu.async_copy(
            val_hbm.at[plsc.Indices(idxv_v.at[sl], ignored_value=IGNORED)],
            bufs[bi], gv_sems[bi])
        return gm, gv

    gm = [None] * NCHUNK
    gv = [None] * NCHUNK
    st = [None] * NCHUNK
    for c in range(min(NBUF, NCHUNK)):
        gm[c], gv[c] = start_gathers(c)
    for c in range(NCHUNK):
        bi = c % NBUF
        gm[c].wait()
        gv[c].wait()
        st[c] = pltpu.async_copy(
            bufs[bi], out_hbm.at[pl.ds(base + c * C, C)], st_sems[bi])
        nxt = c + NBUF
        if nxt < NCHUNK:
            st[c].wait()
            gm[nxt], gv[nxt] = start_gathers(nxt)
    for c in range(max(0, NCHUNK - NBUF), NCHUNK):
        st[c].wait()


@functools.partial(jax.jit, static_argnames=())
def kernel(mem, val, step, batch_idx):
    mem_flat = mem.reshape(TB, D)
    step = jnp.asarray(step, dtype=jnp.int32)
    lo_arr = jnp.full((LANES,), step * B, dtype=jnp.int32)

    run = pl.kernel(
        _body,
        out_type=jax.ShapeDtypeStruct((MB_ROWS, D), jnp.float32),
        mesh=plsc.VectorSubcoreMesh(core_axis_name="c", subcore_axis_name="s"),
        scratch_types=[
            pltpu.VMEM((N_PER_W,), jnp.int32),
            pltpu.VMEM((N_PER_W,), jnp.int32),
            pltpu.VMEM((LANES,), jnp.int32),
            pltpu.VMEM((C, D), jnp.float32),
            pltpu.VMEM((C, D), jnp.float32),
            pltpu.VMEM((C, D), jnp.float32),
        ] + [pltpu.SemaphoreType.DMA] * 9,
    )
    return run(mem_flat, val, lo_arr, batch_idx.astype(jnp.int32))
